# baseline (speedup 1.0000x reference)
import functools

import jax
import jax.numpy as jnp
from jax import lax
from jax.experimental import pallas as pl
from jax.experimental.pallas import tpu as pltpu
from jax.experimental.pallas import tpu_sc as plsc

D = 1024
DH = 1024
E = 8
F = 2048
R = 16
ER = E * R

S_BLK = 512
S_BLK_G = 2048
LN_EPS = 1e-5

NW = 32
LANES = 16


def _gate_kernel(x_ref, hist_ref, persona_ref, W1_ref, b1_ref, W2_ref, b2_ref,
                 lng_ref, lnb_ref, gw_ref, gb_ref, logits_ref):
    xb = x_ref[...]
    W1 = W1_ref[...]
    W1x = W1[:, :D]
    W1h = W1[:, D:D + DH]
    W1p = W1[:, D + DH:]

    base1 = (jnp.dot(xb, W1x.T) + jnp.dot(hist_ref[...], W1h.T)
             + b1_ref[...])
    pc = jnp.dot(persona_ref[...], W1p.T)
    W2 = W2_ref[...]
    b2 = b2_ref[...]
    gw = lng_ref[...] * gw_ref[...]
    gw_sum = jnp.sum(gw)
    cterm = jnp.sum(lnb_ref[...] * gw_ref[...]) + gb_ref[0, 0]

    cols = []
    for e in range(E):
        h1 = jax.nn.relu(base1 + pc[e][None, :])
        h2 = jax.nn.relu(jnp.dot(h1, W2.T) + b2)
        m = jnp.mean(h2, axis=1, keepdims=True)
        v = jnp.mean((h2 - m) ** 2, axis=1, keepdims=True)
        rstd = jax.lax.rsqrt(v + LN_EPS)
        lg = (jnp.dot(h2, gw.T) - m * gw_sum) * rstd + cterm
        cols.append(lg)
    logits = jnp.concatenate(cols, axis=1)
    logits_ref[...] = logits.T


def _route_sc_body(logits_hbm, out_hbm, lv, ov, wid, chunk):
    base = wid * chunk
    pltpu.sync_copy(logits_hbm.at[:, pl.ds(base, chunk)], lv)
    for j in range(chunk // LANES):
        sl = pl.ds(j * LANES, LANES)
        v = [lv[e, sl] for e in range(E)]
        mx = v[0]
        for e in range(1, E):
            mx = jnp.maximum(mx, v[e])
        s = jnp.zeros((LANES,), jnp.float32)
        pe = []
        for e in range(E):
            t = jnp.exp(v[e] - mx)
            pe.append(t)
            s = s + t
        inv = 1.0 / s
        best = pe[0]
        bi = jnp.zeros((LANES,), jnp.int32)
        for e in range(1, E):
            gt = pe[e] > best
            best = jnp.where(gt, pe[e], best)
            bi = jnp.where(gt, e, bi)
        sbest = jnp.full((LANES,), -1.0, jnp.float32)
        si = jnp.zeros((LANES,), jnp.int32)
        for e in range(E):
            cand = (bi != e) & (pe[e] > sbest)
            sbest = jnp.where(cand, pe[e], sbest)
            si = jnp.where(cand, e, si)
        p1 = best * inv
        p2 = sbest * inv
        zero = jnp.zeros((LANES,), jnp.float32)
        one = jnp.ones((LANES,), jnp.float32)
        for e in range(E):
            m1 = bi == e
            m2 = si == e
            ov[e, sl] = jnp.where(m1, one, zero)
            ov[E + e, sl] = jnp.where(m2, one, zero)
            ov[2 * E + e, sl] = jnp.where(m1, p1, zero)
            ov[3 * E + e, sl] = jnp.where(m2, p2, zero)
    pltpu.sync_copy(ov, out_hbm.at[:, pl.ds(base, chunk)])


def _route_sc_kernel(logits_hbm, out_hbm, lv, ov):
    wid = lax.axis_index("s") * 2 + lax.axis_index("c")
    chunk = 128
    n_used = logits_hbm.shape[1] // chunk

    @pl.when(wid < n_used)
    def _():
        _route_sc_body(logits_hbm, out_hbm, lv, ov, wid, chunk)


def _expert_kernel(x_ref, route_ref, Wg_ref, Wu_ref, Wd_ref,
                   AgF_ref, BgF_ref, AuF_ref, BuF_ref, AdF_ref, BdF_ref,
                   out_ref):
    xb = x_ref[...]
    route = route_ref[...]
    G = jnp.dot(xb, Wg_ref[...].T)
    U = jnp.dot(xb, Wu_ref[...].T)
    zg = jnp.dot(xb, AgF_ref[...].T)
    zu = jnp.dot(xb, AuF_ref[...].T)

    exp_e = jax.lax.broadcasted_iota(jnp.int32, (E, ER), 0)
    exp_c = jax.lax.broadcasted_iota(jnp.int32, (E, ER), 1)
    expand = (exp_c // R == exp_e).astype(jnp.float32)
    cdim = (((0,), (0,)), ((), ()))
    ones_col = jnp.ones((E, 1), jnp.float32)

    acc_a = jnp.zeros((S_BLK, F), jnp.float32)
    acc_y = jnp.zeros((S_BLK, ER), jnp.float32)
    for k in range(2):
        oh = route[k * E:(k + 1) * E]
        wm = route[(2 + k) * E:(3 + k) * E]
        mask = lax.dot_general(oh, expand, cdim)
        w = lax.dot_general(wm, ones_col, cdim)
        wmask = lax.dot_general(wm, expand, cdim)
        g = G + jnp.dot(zg * mask, BgF_ref[...].T)
        u = U + jnp.dot(zu * mask, BuF_ref[...].T)
        a = g * jax.lax.logistic(g) * u
        acc_a = acc_a + w * a
        ya = jnp.dot(a, AdF_ref[...].T)
        acc_y = acc_y + ya * wmask

    out_ref[...] = (jnp.dot(acc_a, Wd_ref[...].T)
                    + jnp.dot(acc_y, BdF_ref[...].T))


def kernel(x, history_hidden_embedding, persona_embedding, W1, b1, W2, b2,
           ln_g, ln_b, gate_w, gate_b, Wg, Wu, Wd, Ag, Bg, Au, Bu, Ad, Bd):
    B, S, _ = x.shape
    xf = x.reshape(B * S, D)
    n_blk = (B * S) // S_BLK

    AgF = Ag.reshape(ER, D)
    AuF = Au.reshape(ER, D)
    AdF = Ad.reshape(ER, F)
    BgF = jnp.transpose(Bg, (1, 0, 2)).reshape(F, ER)
    BuF = jnp.transpose(Bu, (1, 0, 2)).reshape(F, ER)
    BdF = jnp.transpose(Bd, (1, 0, 2)).reshape(D, ER)

    inv = lambda shape: pl.BlockSpec(shape, lambda i: (0,) * len(shape))

    logits_t = pl.pallas_call(
        _gate_kernel,
        grid=((B * S) // S_BLK_G,),
        in_specs=[
            pl.BlockSpec((S_BLK_G, D), lambda i: (i, 0)),
            inv((1, DH)),
            inv((E, D)),
            inv((128, D + DH + D)),
            inv((1, 128)),
            inv((D, 128)),
            inv((1, D)),
            inv((1, D)),
            inv((1, D)),
            inv((1, D)),
            inv((1, 1)),
        ],
        out_specs=pl.BlockSpec((E, S_BLK_G), lambda i: (0, i)),
        out_shape=jax.ShapeDtypeStruct((E, B * S), jnp.float32),
    )(xf, history_hidden_embedding, persona_embedding, W1,
      b1.reshape(1, 128), W2, b2.reshape(1, D), ln_g.reshape(1, D),
      ln_b.reshape(1, D), gate_w, gate_b.reshape(1, 1))

    route = pl.kernel(
        _route_sc_kernel,
        mesh=plsc.VectorSubcoreMesh(core_axis_name="c", subcore_axis_name="s"),
        out_type=jax.ShapeDtypeStruct((4 * E, B * S), jnp.float32),
        scratch_types=[
            pltpu.VMEM((E, 128), jnp.float32),
            pltpu.VMEM((4 * E, 128), jnp.float32),
        ],
    )(logits_t)

    out = pl.pallas_call(
        _expert_kernel,
        grid=(n_blk,),
        in_specs=[
            pl.BlockSpec((S_BLK, D), lambda i: (i, 0)),
            pl.BlockSpec((4 * E, S_BLK), lambda i: (0, i)),
            inv((F, D)),
            inv((F, D)),
            inv((D, F)),
            inv((ER, D)),
            inv((F, ER)),
            inv((ER, D)),
            inv((F, ER)),
            inv((ER, F)),
            inv((D, ER)),
        ],
        out_specs=pl.BlockSpec((S_BLK, D), lambda i: (i, 0)),
        out_shape=jax.ShapeDtypeStruct((B * S, D), jnp.float32),
    )(xf, route, Wg, Wu, Wd, AgF, BgF, AuF, BuF, AdF, BdF)

    return out.reshape(B, S, D)


# speedup vs baseline: 1.0372x; 1.0372x over previous
import functools

import jax
import jax.numpy as jnp
from jax import lax
from jax.experimental import pallas as pl
from jax.experimental.pallas import tpu as pltpu
from jax.experimental.pallas import tpu_sc as plsc

D = 1024
DH = 1024
E = 8
F = 2048
R = 16
ER = E * R

S_BLK = 512
S_BLK_G = 1024
LN_EPS = 1e-5

NW = 32
LANES = 16


def _gate_kernel(x_ref, hist_ref, persona_ref, W1_ref, b1_ref, W2_ref, b2_ref,
                 lng_ref, lnb_ref, gw_ref, gb_ref, logits_ref):
    xb = x_ref[...]
    W1 = W1_ref[...]
    W1x = W1[:, :D]
    W1h = W1[:, D:D + DH]
    W1p = W1[:, D + DH:]

    base1 = (jnp.dot(xb, W1x.T) + jnp.dot(hist_ref[...], W1h.T)
             + b1_ref[...])
    pc = jnp.dot(persona_ref[...], W1p.T)
    W2 = W2_ref[...]
    b2 = b2_ref[...]
    gw = lng_ref[...] * gw_ref[...]
    gw_sum = jnp.sum(gw)
    cterm = jnp.sum(lnb_ref[...] * gw_ref[...]) + gb_ref[0, 0]

    cols = []
    for e in range(E):
        h1 = jax.nn.relu(base1 + pc[e][None, :])
        h2 = jax.nn.relu(jnp.dot(h1, W2.T) + b2)
        m = jnp.mean(h2, axis=1, keepdims=True)
        v = jnp.mean((h2 - m) ** 2, axis=1, keepdims=True)
        rstd = jax.lax.rsqrt(v + LN_EPS)
        lg = (jnp.dot(h2, gw.T) - m * gw_sum) * rstd + cterm
        cols.append(lg)
    logits = jnp.concatenate(cols, axis=1)
    logits_ref[...] = logits.T


def _route_sc_body(logits_hbm, out_hbm, lv, ov, wid, chunk):
    base = wid * chunk
    pltpu.sync_copy(logits_hbm.at[:, pl.ds(base, chunk)], lv)
    for j in range(chunk // LANES):
        sl = pl.ds(j * LANES, LANES)
        v = [lv[e, sl] for e in range(E)]
        mx = v[0]
        for e in range(1, E):
            mx = jnp.maximum(mx, v[e])
        s = jnp.zeros((LANES,), jnp.float32)
        pe = []
        for e in range(E):
            t = jnp.exp(v[e] - mx)
            pe.append(t)
            s = s + t
        inv = 1.0 / s
        best = pe[0]
        bi = jnp.zeros((LANES,), jnp.int32)
        for e in range(1, E):
            gt = pe[e] > best
            best = jnp.where(gt, pe[e], best)
            bi = jnp.where(gt, e, bi)
        sbest = jnp.full((LANES,), -1.0, jnp.float32)
        si = jnp.zeros((LANES,), jnp.int32)
        for e in range(E):
            cand = (bi != e) & (pe[e] > sbest)
            sbest = jnp.where(cand, pe[e], sbest)
            si = jnp.where(cand, e, si)
        p1 = best * inv
        p2 = sbest * inv
        zero = jnp.zeros((LANES,), jnp.float32)
        one = jnp.ones((LANES,), jnp.float32)
        for e in range(E):
            m1 = bi == e
            m2 = si == e
            ov[e, sl] = jnp.where(m1, one, zero)
            ov[E + e, sl] = jnp.where(m2, one, zero)
            ov[2 * E + e, sl] = jnp.where(m1, p1, zero)
            ov[3 * E + e, sl] = jnp.where(m2, p2, zero)
    pltpu.sync_copy(ov, out_hbm.at[:, pl.ds(base, chunk)])


def _route_sc_kernel(logits_hbm, out_hbm, lv, ov):
    wid = lax.axis_index("s") * 2 + lax.axis_index("c")
    chunk = 128
    n_used = logits_hbm.shape[1] // chunk

    @pl.when(wid < n_used)
    def _():
        _route_sc_body(logits_hbm, out_hbm, lv, ov, wid, chunk)


def _expert_kernel(x_ref, route_ref, Wguz_ref, Bgu_ref, AdF_ref, Wdd_ref,
                   out_ref):
    xb = x_ref[...]
    route = route_ref[...]
    GUZ = jnp.dot(xb, Wguz_ref[...].T)
    G = GUZ[:, :F]
    U = GUZ[:, F:2 * F]
    z = GUZ[:, 2 * F:]

    exp_e = jax.lax.broadcasted_iota(jnp.int32, (E, 2 * ER), 0)
    exp_c = jax.lax.broadcasted_iota(jnp.int32, (E, 2 * ER), 1)
    expand2 = ((exp_c // R) % E == exp_e).astype(jnp.float32)
    cdim = (((0,), (0,)), ((), ()))
    ones_col = jnp.ones((E, 1), jnp.float32)
    exp_y_e = jax.lax.broadcasted_iota(jnp.int32, (E, ER), 0)
    exp_y_c = jax.lax.broadcasted_iota(jnp.int32, (E, ER), 1)
    expand_y = (exp_y_c // R == exp_y_e).astype(jnp.float32)

    acc_a = jnp.zeros((S_BLK, F), jnp.float32)
    acc_y = jnp.zeros((S_BLK, ER), jnp.float32)
    for k in range(2):
        oh = route[k * E:(k + 1) * E]
        wm = route[(2 + k) * E:(3 + k) * E]
        mask2 = lax.dot_general(oh, expand2, cdim)
        w = lax.dot_general(wm, ones_col, cdim)
        wmask = lax.dot_general(wm, expand_y, cdim)
        dgu = jnp.dot(z * mask2, Bgu_ref[...].T)
        g = G + dgu[:, :F]
        u = U + dgu[:, F:]
        a = g * jax.lax.logistic(g) * u
        acc_a = acc_a + w * a
        ya = jnp.dot(a, AdF_ref[...].T)
        acc_y = acc_y + ya * wmask

    acc = jnp.concatenate([acc_a, acc_y], axis=1)
    out_ref[...] = jnp.dot(acc, Wdd_ref[...].T)


def kernel(x, history_hidden_embedding, persona_embedding, W1, b1, W2, b2,
           ln_g, ln_b, gate_w, gate_b, Wg, Wu, Wd, Ag, Bg, Au, Bu, Ad, Bd):
    B, S, _ = x.shape
    xf = x.reshape(B * S, D)
    n_blk = (B * S) // S_BLK

    AgF = Ag.reshape(ER, D)
    AuF = Au.reshape(ER, D)
    AdF = Ad.reshape(ER, F)
    BgF = jnp.transpose(Bg, (1, 0, 2)).reshape(F, ER)
    BuF = jnp.transpose(Bu, (1, 0, 2)).reshape(F, ER)
    BdF = jnp.transpose(Bd, (1, 0, 2)).reshape(D, ER)
    Wguz = jnp.concatenate([Wg, Wu, AgF, AuF], axis=0)
    zpad = jnp.zeros((F, ER), jnp.float32)
    Bgu = jnp.concatenate([
        jnp.concatenate([BgF, zpad], axis=1),
        jnp.concatenate([zpad, BuF], axis=1),
    ], axis=0)
    Wdd = jnp.concatenate([Wd, BdF], axis=1)

    inv = lambda shape: pl.BlockSpec(shape, lambda i: (0,) * len(shape))

    logits_t = pl.pallas_call(
        _gate_kernel,
        grid=((B * S) // S_BLK_G,),
        in_specs=[
            pl.BlockSpec((S_BLK_G, D), lambda i: (i, 0)),
            inv((1, DH)),
            inv((E, D)),
            inv((128, D + DH + D)),
            inv((1, 128)),
            inv((D, 128)),
            inv((1, D)),
            inv((1, D)),
            inv((1, D)),
            inv((1, D)),
            inv((1, 1)),
        ],
        out_specs=pl.BlockSpec((E, S_BLK_G), lambda i: (0, i)),
        out_shape=jax.ShapeDtypeStruct((E, B * S), jnp.float32),
    )(xf, history_hidden_embedding, persona_embedding, W1,
      b1.reshape(1, 128), W2, b2.reshape(1, D), ln_g.reshape(1, D),
      ln_b.reshape(1, D), gate_w, gate_b.reshape(1, 1))

    route = pl.kernel(
        _route_sc_kernel,
        mesh=plsc.VectorSubcoreMesh(core_axis_name="c", subcore_axis_name="s"),
        out_type=jax.ShapeDtypeStruct((4 * E, B * S), jnp.float32),
        scratch_types=[
            pltpu.VMEM((E, 128), jnp.float32),
            pltpu.VMEM((4 * E, 128), jnp.float32),
        ],
    )(logits_t)

    out = pl.pallas_call(
        _expert_kernel,
        grid=(n_blk,),
        in_specs=[
            pl.BlockSpec((S_BLK, D), lambda i: (i, 0)),
            pl.BlockSpec((4 * E, S_BLK), lambda i: (0, i)),
            inv((2 * F + 2 * ER, D)),
            inv((2 * F, 2 * ER)),
            inv((ER, F)),
            inv((D, F + ER)),
        ],
        out_specs=pl.BlockSpec((S_BLK, D), lambda i: (i, 0)),
        out_shape=jax.ShapeDtypeStruct((B * S, D), jnp.float32),
    )(xf, route, Wguz, Bgu, AdF, Wdd)

    return out.reshape(B, S, D)


# speedup vs baseline: 1.0456x; 1.0081x over previous
import jax
import jax.numpy as jnp
from jax import lax
from jax.experimental import pallas as pl
from jax.experimental.pallas import tpu as pltpu
from jax.experimental.pallas import tpu_sc as plsc

D = 1024
DH = 1024
E = 8
F = 2048
R = 16
ER = E * R

S_BLK = 512
S_BLK_G = 1024
LN_EPS = 1e-5

LANES = 16


def _gate_kernel(x_ref, hist_ref, persona_ref, W1_ref, b1_ref, W2_ref, b2_ref,
                 lng_ref, lnb_ref, gw_ref, gb_ref, logits_ref):
    xb = x_ref[...]
    W1 = W1_ref[...]
    W1x = W1[:, :D]
    W1h = W1[:, D:D + DH]
    W1p = W1[:, D + DH:]

    base1 = (jnp.dot(xb, W1x.T) + jnp.dot(hist_ref[...], W1h.T)
             + b1_ref[...])
    pc = jnp.dot(persona_ref[...], W1p.T)
    W2 = W2_ref[...]
    b2 = b2_ref[...]
    gw = lng_ref[...] * gw_ref[...]
    gw_sum = jnp.sum(gw)
    cterm = jnp.sum(lnb_ref[...] * gw_ref[...]) + gb_ref[0, 0]

    cols = []
    for e in range(E):
        h1 = jax.nn.relu(base1 + pc[e][None, :])
        h2 = jax.nn.relu(jnp.dot(h1, W2.T) + b2)
        m = jnp.mean(h2, axis=1, keepdims=True)
        v = jnp.mean((h2 - m) ** 2, axis=1, keepdims=True)
        rstd = jax.lax.rsqrt(v + LN_EPS)
        lg = (jnp.dot(h2, gw.T) - m * gw_sum) * rstd + cterm
        cols.append(lg)
    logits = jnp.concatenate(cols, axis=1)
    logits_ref[...] = logits.T


def _route_sc_body(logits_hbm, out_hbm, lv, ov, wid, chunk):
    base = wid * chunk
    pltpu.sync_copy(logits_hbm.at[:, pl.ds(base, chunk)], lv)
    for j in range(chunk // LANES):
        sl = pl.ds(j * LANES, LANES)
        v = [lv[e, sl] for e in range(E)]
        mx = v[0]
        for e in range(1, E):
            mx = jnp.maximum(mx, v[e])
        s = jnp.zeros((LANES,), jnp.float32)
        pe = []
        for e in range(E):
            t = jnp.exp(v[e] - mx)
            pe.append(t)
            s = s + t
        inv = 1.0 / s
        best = pe[0]
        bi = jnp.zeros((LANES,), jnp.int32)
        for e in range(1, E):
            gt = pe[e] > best
            best = jnp.where(gt, pe[e], best)
            bi = jnp.where(gt, e, bi)
        sbest = jnp.full((LANES,), -1.0, jnp.float32)
        si = jnp.zeros((LANES,), jnp.int32)
        for e in range(E):
            cand = (bi != e) & (pe[e] > sbest)
            sbest = jnp.where(cand, pe[e], sbest)
            si = jnp.where(cand, e, si)
        p1 = best * inv
        p2 = sbest * inv
        zero = jnp.zeros((LANES,), jnp.float32)
        one = jnp.ones((LANES,), jnp.float32)
        for e in range(E):
            m1 = bi == e
            m2 = si == e
            ov[e, sl] = jnp.where(m1, one, zero)
            ov[E + e, sl] = jnp.where(m2, one, zero)
            ov[2 * E + e, sl] = jnp.where(m1, p1, zero)
            ov[3 * E + e, sl] = jnp.where(m2, p2, zero)
    pltpu.sync_copy(ov, out_hbm.at[:, pl.ds(base, chunk)])


def _route_sc_kernel(logits_hbm, out_hbm, lv, ov):
    wid = lax.axis_index("s") * 2 + lax.axis_index("c")
    chunk = 128
    n_used = logits_hbm.shape[1] // chunk

    @pl.when(wid < n_used)
    def _():
        _route_sc_body(logits_hbm, out_hbm, lv, ov, wid, chunk)


def _expert_kernel(x_ref, route_ref, Wg_ref, Wu_ref, Wd_ref,
                   AgF_ref, BgF_ref, AuF_ref, BuF_ref, AdF_ref, BdF_ref,
                   out_ref):
    xb = x_ref[...]
    route = route_ref[...]
    G = jnp.dot(xb, Wg_ref[...].T)
    U = jnp.dot(xb, Wu_ref[...].T)
    zg = jnp.dot(xb, AgF_ref[...].T)
    zu = jnp.dot(xb, AuF_ref[...].T)

    exp_e = jax.lax.broadcasted_iota(jnp.int32, (E, ER), 0)
    exp_c = jax.lax.broadcasted_iota(jnp.int32, (E, ER), 1)
    expand = (exp_c // R == exp_e).astype(jnp.float32)
    cdim = (((0,), (0,)), ((), ()))
    ones_col = jnp.ones((E, 1), jnp.float32)

    acc_a = jnp.zeros((S_BLK, F), jnp.float32)
    acc_y = jnp.zeros((S_BLK, ER), jnp.float32)
    for k in range(2):
        oh = route[k * E:(k + 1) * E]
        wm = route[(2 + k) * E:(3 + k) * E]
        mask = lax.dot_general(oh, expand, cdim)
        w = lax.dot_general(wm, ones_col, cdim)
        wmask = lax.dot_general(wm, expand, cdim)
        g = G + jnp.dot(zg * mask, BgF_ref[...].T)
        u = U + jnp.dot(zu * mask, BuF_ref[...].T)
        a = g * jax.lax.logistic(g) * u
        acc_a = acc_a + w * a
        ya = jnp.dot(a, AdF_ref[...].T)
        acc_y = acc_y + ya * wmask

    out_ref[...] = (jnp.dot(acc_a, Wd_ref[...].T)
                    + jnp.dot(acc_y, BdF_ref[...].T))


def kernel(x, history_hidden_embedding, persona_embedding, W1, b1, W2, b2,
           ln_g, ln_b, gate_w, gate_b, Wg, Wu, Wd, Ag, Bg, Au, Bu, Ad, Bd):
    B, S, _ = x.shape
    xf = x.reshape(B * S, D)
    n_blk = (B * S) // S_BLK

    AgF = Ag.reshape(ER, D)
    AuF = Au.reshape(ER, D)
    AdF = Ad.reshape(ER, F)
    BgF = jnp.transpose(Bg, (1, 0, 2)).reshape(F, ER)
    BuF = jnp.transpose(Bu, (1, 0, 2)).reshape(F, ER)
    BdF = jnp.transpose(Bd, (1, 0, 2)).reshape(D, ER)

    inv = lambda shape: pl.BlockSpec(shape, lambda i: (0,) * len(shape))

    logits_t = pl.pallas_call(
        _gate_kernel,
        grid=((B * S) // S_BLK_G,),
        in_specs=[
            pl.BlockSpec((S_BLK_G, D), lambda i: (i, 0)),
            inv((1, DH)),
            inv((E, D)),
            inv((128, D + DH + D)),
            inv((1, 128)),
            inv((D, 128)),
            inv((1, D)),
            inv((1, D)),
            inv((1, D)),
            inv((1, D)),
            inv((1, 1)),
        ],
        out_specs=pl.BlockSpec((E, S_BLK_G), lambda i: (0, i)),
        out_shape=jax.ShapeDtypeStruct((E, B * S), jnp.float32),
    )(xf, history_hidden_embedding, persona_embedding, W1,
      b1.reshape(1, 128), W2, b2.reshape(1, D), ln_g.reshape(1, D),
      ln_b.reshape(1, D), gate_w, gate_b.reshape(1, 1))

    route = pl.kernel(
        _route_sc_kernel,
        mesh=plsc.VectorSubcoreMesh(core_axis_name="c", subcore_axis_name="s"),
        out_type=jax.ShapeDtypeStruct((4 * E, B * S), jnp.float32),
        scratch_types=[
            pltpu.VMEM((E, 128), jnp.float32),
            pltpu.VMEM((4 * E, 128), jnp.float32),
        ],
    )(logits_t)

    out = pl.pallas_call(
        _expert_kernel,
        grid=(n_blk,),
        in_specs=[
            pl.BlockSpec((S_BLK, D), lambda i: (i, 0)),
            pl.BlockSpec((4 * E, S_BLK), lambda i: (0, i)),
            inv((F, D)),
            inv((F, D)),
            inv((D, F)),
            inv((ER, D)),
            inv((F, ER)),
            inv((ER, D)),
            inv((F, ER)),
            inv((ER, F)),
            inv((D, ER)),
        ],
        out_specs=pl.BlockSpec((S_BLK, D), lambda i: (i, 0)),
        out_shape=jax.ShapeDtypeStruct((B * S, D), jnp.float32),
    )(xf, route, Wg, Wu, Wd, AgF, BgF, AuF, BuF, AdF, BdF)

    return out.reshape(B, S, D)
